# baseline (speedup 1.0000x reference)
import functools

import jax
import jax.numpy as jnp
from jax import lax
from jax.experimental import pallas as pl
from jax.experimental.pallas import tpu as pltpu
from jax.experimental.pallas import tpu_sc as plsc
from jax._src.pallas import mpmd as plmpmd

BATCH = 4
SEQ = 8192
EMBED = 1024

NC = 2
NS = 16
NW = NC * NS
B = BATCH * SEQ

R_SCS = 2048
R_SCS_PER = R_SCS // NC
R_TEC = B - R_SCS
BPW = R_TEC // NW
CHUNK = 8
NCH = BPW // CHUNK
NBUF = 8
NGRP = NCH // NBUF
SW = 8

_vec_mesh = plsc.VectorSubcoreMesh(core_axis_name="c", subcore_axis_name="s")
_sca_mesh = plsc.ScalarSubcoreMesh(axis_name="c")


def _tec_fn(table_hbm, idx_tec_hbm, idx_scs_hbm, out_hbm,
            idx_v, bufs, gsems, idx_smem, dsem):
    del idx_scs_hbm, idx_smem, dsem
    wid = lax.axis_index("s") * NC + lax.axis_index("c")
    pltpu.sync_copy(idx_tec_hbm.at[wid], idx_v)

    def out_slice(g):
        return out_hbm.at[pl.ds(wid * BPW + g * CHUNK, CHUNK)]

    for b in range(NBUF):
        pltpu.async_copy(table_hbm.at[idx_v.at[b]], bufs[b], gsems[b])

    def group(i, carry):
        for b in range(NBUF):
            g = i * NBUF + b
            pltpu.make_async_copy(table_hbm.at[idx_v.at[g]], bufs[b], gsems[b]).wait()
            pltpu.sync_copy(bufs[b], out_slice(g))
            pltpu.async_copy(table_hbm.at[idx_v.at[g + NBUF]], bufs[b], gsems[b])
        return carry

    lax.fori_loop(0, NGRP - 1, group, 0)

    for b in range(NBUF):
        g = (NGRP - 1) * NBUF + b
        pltpu.make_async_copy(table_hbm.at[idx_v.at[g]], bufs[b], gsems[b]).wait()
        pltpu.sync_copy(bufs[b], out_slice(g))


def _scs_fn(table_hbm, idx_tec_hbm, idx_scs_hbm, out_hbm,
            idx_v, bufs, gsems, idx_smem, dsem):
    del idx_tec_hbm, idx_v, bufs, gsems
    c = lax.axis_index("c")
    pltpu.sync_copy(idx_scs_hbm.at[pl.ds(c * R_SCS_PER, R_SCS_PER)], idx_smem)
    base_out = R_TEC + c * R_SCS_PER

    def batch(bi, carry):
        copies = []
        for j in range(SW):
            i = bi * SW + j
            r = idx_smem[i]
            copies.append(
                pltpu.async_copy(table_hbm.at[r], out_hbm.at[base_out + i], dsem))
        for cp in copies:
            cp.wait()
        return carry

    lax.fori_loop(0, R_SCS_PER // SW, batch, 0)


_gather_kernel = plmpmd.mpmd_map(
    [(_sca_mesh, _scs_fn), (_vec_mesh, _tec_fn)],
    out_types=jax.ShapeDtypeStruct((B, EMBED), jnp.float32),
    scratch_types=(
        (pltpu.VMEM @ _vec_mesh)((NCH, CHUNK), jnp.int32),
        tuple((pltpu.VMEM @ _vec_mesh)((CHUNK, EMBED), jnp.float32)
              for _ in range(NBUF)),
        tuple(pltpu.SemaphoreType.DMA @ _vec_mesh for _ in range(NBUF)),
        (pltpu.SMEM @ _sca_mesh)((R_SCS_PER,), jnp.int32),
        pltpu.SemaphoreType.DMA @ _sca_mesh,
    ),
)


def kernel(position_ids, table):
    idx = position_ids.reshape(-1).astype(jnp.int32)
    idx_tec = idx[:R_TEC].reshape(NW, NCH, CHUNK)
    idx_scs = idx[R_TEC:]
    out = _gather_kernel(table, idx_tec, idx_scs)
    return out.reshape(BATCH, SEQ, EMBED)


# speedup vs baseline: 2.7930x; 2.7930x over previous
import functools

import jax
import jax.numpy as jnp
from jax import lax
from jax.experimental import pallas as pl
from jax.experimental.pallas import tpu as pltpu
from jax.experimental.pallas import tpu_sc as plsc

BATCH = 4
SEQ = 8192
EMBED = 1024

NC = 2
NS = 16
NW = NC * NS
B = BATCH * SEQ
B_PER_W = B // NW
CHUNK = 8
NCH = B_PER_W // CHUNK
NBUF = 8
NGRP = NCH // NBUF

_mesh = plsc.VectorSubcoreMesh(core_axis_name="c", subcore_axis_name="s")


@functools.partial(
    pl.kernel,
    out_type=jax.ShapeDtypeStruct((NW, NCH, CHUNK, EMBED), jnp.float32),
    mesh=_mesh,
    scratch_types=[
        pltpu.VMEM((NCH, CHUNK), jnp.int32),
        [pltpu.VMEM((CHUNK, EMBED), jnp.float32) for _ in range(NBUF)],
        [pltpu.SemaphoreType.DMA for _ in range(NBUF)],
    ],
)
def _gather_kernel(table_hbm, idx_hbm, out_hbm, idx_v, bufs, gsems):
    wid = lax.axis_index("s") * NC + lax.axis_index("c")
    pltpu.sync_copy(idx_hbm.at[wid], idx_v)

    for b in range(NBUF):
        pltpu.async_copy(table_hbm.at[idx_v.at[b]], bufs[b], gsems[b])

    def group(i, carry):
        for b in range(NBUF):
            g = i * NBUF + b
            pltpu.make_async_copy(table_hbm.at[idx_v.at[g]], bufs[b], gsems[b]).wait()
            pltpu.sync_copy(bufs[b], out_hbm.at[wid, g])
            pltpu.async_copy(table_hbm.at[idx_v.at[g + NBUF]], bufs[b], gsems[b])
        return carry

    lax.fori_loop(0, NGRP - 1, group, 0)

    for b in range(NBUF):
        g = (NGRP - 1) * NBUF + b
        pltpu.make_async_copy(table_hbm.at[idx_v.at[g]], bufs[b], gsems[b]).wait()
        pltpu.sync_copy(bufs[b], out_hbm.at[wid, g])


def kernel(position_ids, table):
    idx = position_ids.reshape(NW, NCH, CHUNK).astype(jnp.int32)
    out = _gather_kernel(table, idx)
    return out.reshape(BATCH, SEQ, EMBED)
